# baseline (speedup 1.0000x reference)
import jax
import jax.numpy as jnp
from jax import lax
from jax.experimental import pallas as pl
from jax.experimental.pallas import tpu as pltpu
from jax.experimental.pallas import tpu_sc as plsc

VOCAB = 1_000_000
MAX_LEN = 200
EMBED_DIM = 64
BATCH = 4096

NC, NS, L = 2, 16, 16
NW = NC * NS
BW = BATCH // NW
LC = 2
NCHUNK = MAX_LEN // LC
KB = BW // L
DHI = EMBED_DIM // 8


def _body(x_hbm, tok_hbm, pos_hbm, out_hbm, pos_v, idx_v, rows_v, obuf,
          isems, gsems, wsems):
    wid = lax.axis_index("s") * NC + lax.axis_index("c")
    b0 = wid * BW
    pltpu.sync_copy(pos_hbm, pos_v)
    iota = lax.iota(jnp.int32, L)

    def idx_fetch(g, p):
        return pltpu.make_async_copy(
            x_hbm.at[pl.ds(g * LC, LC), pl.ds(b0, BW)], idx_v.at[p], isems[p])

    def gathers(g, p):
        del g
        return [pltpu.make_async_copy(
            tok_hbm.at[idx_v.at[p, i]], rows_v.at[p, i], gsems[p])
            for i in range(LC)]

    def write(g, p):
        return pltpu.make_async_copy(
            obuf.at[p], out_hbm.at[pl.ds(g * LC, LC), :, wid], wsems[p])

    idx_fetch(0, 0).start()
    idx_fetch(1, 1).start()
    idx_fetch(0, 0).wait()
    for c in gathers(0, 0):
        c.start()

    @pl.loop(0, NCHUNK, step=2)
    def _ring(g0):
        for par in range(2):
            g = g0 + par

            @pl.when(g + 1 < NCHUNK)
            def _launch_next():
                idx_fetch(g + 1, par ^ 1).wait()
                for c in gathers(g + 1, par ^ 1):
                    c.start()

            @pl.when(g >= 2)
            def _drain_write():
                write(g - 2, par).wait()

            for c in gathers(g, par):
                c.wait()

            @pl.when(g + 2 < NCHUNK)
            def _prefetch_idx():
                idx_fetch(g + 2, par).start()

            for i in range(LC):
                lpos = g * LC + i

                @pl.loop(0, EMBED_DIM)
                def _cols(d):
                    dvec = iota * 0 + d
                    pv = plsc.load_gather(
                        pos_v, [iota * 0 + lpos, dvec])
                    d_hi = d >> 3
                    d_lo = d & 7
                    for k in range(KB):
                        vals = plsc.load_gather(
                            rows_v.at[par, i], [iota + (k * L), dvec])
                        obuf[par, i, d_hi, d_lo, pl.ds(k * L, L)] = vals + pv

            write(g, par).start()

    write(NCHUNK - 2, 0).wait()
    write(NCHUNK - 1, 1).wait()


def kernel(x, token_table, pos_table):
    xt = x.T.astype(jnp.int32)
    out5 = pl.kernel(
        _body,
        out_type=jax.ShapeDtypeStruct((MAX_LEN, DHI, NW, 8, BW), jnp.float32),
        mesh=plsc.VectorSubcoreMesh(core_axis_name="c", subcore_axis_name="s"),
        compiler_params=pltpu.CompilerParams(use_tc_tiling_on_sc=False,
                                               needs_layout_passes=False),
        scratch_types=[
            pltpu.VMEM((MAX_LEN, EMBED_DIM), jnp.float32),
            pltpu.VMEM((2, LC, BW), jnp.int32),
            pltpu.VMEM((2, LC, BW, EMBED_DIM), jnp.float32),
            pltpu.VMEM((2, LC, DHI, 8, BW), jnp.float32),
            [pltpu.SemaphoreType.DMA] * 2,
            [pltpu.SemaphoreType.DMA] * 2,
            [pltpu.SemaphoreType.DMA] * 2,
        ],
    )(xt, token_table, pos_table)
    return out5.transpose(2, 4, 0, 1, 3).reshape(BATCH, MAX_LEN, EMBED_DIM)


# speedup vs baseline: 1.3177x; 1.3177x over previous
import jax
import jax.numpy as jnp
from jax import lax
from jax.experimental import pallas as pl
from jax.experimental.pallas import tpu as pltpu
from jax.experimental.pallas import tpu_sc as plsc

VOCAB = 1_000_000
MAX_LEN = 200
EMBED_DIM = 64
BATCH = 4096

NC, NS, L = 2, 16, 16
NW = NC * NS
BW = BATCH // NW
LC = 2
NCHUNK = MAX_LEN // LC
KB = BW // L
DHI = EMBED_DIM // 8


def _body(x_hbm, tok_hbm, pos_hbm, out_hbm, pos_v, idx_v, rows_v, obuf,
          isems, gsems, wsems):
    wid = lax.axis_index("s") * NC + lax.axis_index("c")
    b0 = wid * BW
    pltpu.sync_copy(pos_hbm, pos_v)
    iota = lax.iota(jnp.int32, L)

    def idx_fetch(g, p):
        return pltpu.make_async_copy(
            x_hbm.at[pl.ds(g * LC, LC), pl.ds(b0, BW)], idx_v.at[p], isems[p])

    def gathers(g, p):
        del g
        return [pltpu.make_async_copy(
            tok_hbm.at[idx_v.at[p, i]], rows_v.at[p, i], gsems[p])
            for i in range(LC)]

    def write(g, p):
        return pltpu.make_async_copy(
            obuf.at[p], out_hbm.at[pl.ds(g * LC, LC), :, wid], wsems[p])

    idx_fetch(0, 0).start()
    idx_fetch(1, 1).start()
    idx_fetch(0, 0).wait()
    for c in gathers(0, 0):
        c.start()

    @pl.loop(0, NCHUNK, step=2)
    def _ring(g0):
        for par in range(2):
            g = g0 + par

            @pl.when(g + 1 < NCHUNK)
            def _launch_next():
                idx_fetch(g + 1, par ^ 1).wait()
                for c in gathers(g + 1, par ^ 1):
                    c.start()

            @pl.when(g >= 2)
            def _drain_write():
                write(g - 2, par).wait()

            for c in gathers(g, par):
                c.wait()

            @pl.when(g + 2 < NCHUNK)
            def _prefetch_idx():
                idx_fetch(g + 2, par).start()

            for i in range(LC):
                lpos = g * LC + i
                lvec = iota * 0 + lpos

                @plsc.parallel_loop(0, DHI, unroll=2)
                def _cols(d_hi):
                    d8 = d_hi * 8
                    for d_lo in range(8):
                        dvec = iota * 0 + (d8 + d_lo)
                        pv = plsc.load_gather(pos_v, [lvec, dvec])
                        for k in range(KB):
                            vals = plsc.load_gather(
                                rows_v.at[par, i], [iota + (k * L), dvec])
                            obuf[par, i, d_hi, d_lo, pl.ds(k * L, L)] = vals + pv

            write(g, par).start()

    write(NCHUNK - 2, 0).wait()
    write(NCHUNK - 1, 1).wait()


def kernel(x, token_table, pos_table):
    xt = x.T.astype(jnp.int32)
    out5 = pl.kernel(
        _body,
        out_type=jax.ShapeDtypeStruct((MAX_LEN, DHI, NW, 8, BW), jnp.float32),
        mesh=plsc.VectorSubcoreMesh(core_axis_name="c", subcore_axis_name="s"),
        compiler_params=pltpu.CompilerParams(use_tc_tiling_on_sc=False,
                                               needs_layout_passes=False),
        scratch_types=[
            pltpu.VMEM((MAX_LEN, EMBED_DIM), jnp.float32),
            pltpu.VMEM((2, LC, BW), jnp.int32),
            pltpu.VMEM((2, LC, BW, EMBED_DIM), jnp.float32),
            pltpu.VMEM((2, LC, DHI, 8, BW), jnp.float32),
            [pltpu.SemaphoreType.DMA] * 2,
            [pltpu.SemaphoreType.DMA] * 2,
            [pltpu.SemaphoreType.DMA] * 2,
        ],
    )(xt, token_table, pos_table)
    return out5.transpose(2, 4, 0, 1, 3).reshape(BATCH, MAX_LEN, EMBED_DIM)


# speedup vs baseline: 2.5557x; 1.9395x over previous
import jax
import jax.numpy as jnp
from jax import lax
from jax.experimental import pallas as pl
from jax.experimental.pallas import tpu as pltpu
from jax.experimental.pallas import tpu_sc as plsc

VOCAB = 1_000_000
MAX_LEN = 200
EMBED_DIM = 64
BATCH = 4096

NC, NS, L = 2, 16, 16
NW = NC * NS
BW = BATCH // NW
LC = 2
NCHUNK = MAX_LEN // LC
KB = BW // L
DHI = EMBED_DIM // 8
DV = EMBED_DIM // L


def _body(x_hbm, tok_hbm, pos_hbm, out_hbm, pos_v, idx_v, rows_v, obuf,
          isems, gsems, wsems):
    wid = lax.axis_index("s") * NC + lax.axis_index("c")
    b0 = wid * BW
    pltpu.sync_copy(pos_hbm, pos_v)
    iota = lax.iota(jnp.int32, L)

    def idx_fetch(g, p):
        return pltpu.make_async_copy(
            x_hbm.at[pl.ds(g * LC, LC), pl.ds(b0, BW)], idx_v.at[p], isems[p])

    def gathers(g, p):
        del g
        return [pltpu.make_async_copy(
            tok_hbm.at[idx_v.at[p, i]], rows_v.at[p, i], gsems[p])
            for i in range(LC)]

    def write(g, p):
        return pltpu.make_async_copy(
            obuf.at[p, slice(None), slice(None), slice(None), pl.ds(0, BW)],
            out_hbm.at[pl.ds(g * LC, LC), :, wid], wsems[p])

    idx_fetch(0, 0).start()
    idx_fetch(1, 1).start()
    idx_fetch(0, 0).wait()
    for c in gathers(0, 0):
        c.start()

    @pl.loop(0, NCHUNK, step=2)
    def _ring(g0):
        for par in range(2):
            g = g0 + par

            @pl.when(g + 1 < NCHUNK)
            def _launch_next():
                idx_fetch(g + 1, par ^ 1).wait()
                for c in gathers(g + 1, par ^ 1):
                    c.start()

            @pl.when(g >= 2)
            def _drain_write():
                write(g - 2, par).wait()

            for c in gathers(g, par):
                c.wait()

            @pl.when(g + 2 < NCHUNK)
            def _prefetch_idx():
                idx_fetch(g + 2, par).start()

            dhi_vecs = [(iota + q * L) >> 3 for q in range(DV)]
            dlo_vecs = [iota & 7 for _ in range(DV)]
            for i in range(LC):
                lpos = g * LC + i
                pvs = [pos_v[lpos, pl.ds(q * L, L)] for q in range(DV)]

                @plsc.parallel_loop(0, BW, unroll=2)
                def _rows(b):
                    bvec = iota * 0 + b
                    for q in range(DV):
                        vals = rows_v[par, i, b, pl.ds(q * L, L)] + pvs[q]
                        plsc.store_scatter(
                            obuf.at[par, i], [dhi_vecs[q], dlo_vecs[q], bvec],
                            vals)

            write(g, par).start()

    write(NCHUNK - 2, 0).wait()
    write(NCHUNK - 1, 1).wait()


def kernel(x, token_table, pos_table):
    xt = x.T.astype(jnp.int32)
    out5 = pl.kernel(
        _body,
        out_type=jax.ShapeDtypeStruct((MAX_LEN, DHI, NW, 8, BW), jnp.float32),
        mesh=plsc.VectorSubcoreMesh(core_axis_name="c", subcore_axis_name="s"),
        compiler_params=pltpu.CompilerParams(use_tc_tiling_on_sc=False,
                                               needs_layout_passes=False),
        scratch_types=[
            pltpu.VMEM((MAX_LEN, EMBED_DIM), jnp.float32),
            pltpu.VMEM((2, LC, BW), jnp.int32),
            pltpu.VMEM((2, LC, BW, EMBED_DIM), jnp.float32),
            pltpu.VMEM((2, LC, DHI, 8, BW + 1), jnp.float32),
            [pltpu.SemaphoreType.DMA] * 2,
            [pltpu.SemaphoreType.DMA] * 2,
            [pltpu.SemaphoreType.DMA] * 2,
        ],
    )(xt, token_table, pos_table)
    return out5.transpose(2, 4, 0, 1, 3).reshape(BATCH, MAX_LEN, EMBED_DIM)


# speedup vs baseline: 3.2538x; 1.2731x over previous
import jax
import jax.numpy as jnp
from jax import lax
from jax.experimental import pallas as pl
from jax.experimental.pallas import tpu as pltpu
from jax.experimental.pallas import tpu_sc as plsc

VOCAB = 1_000_000
MAX_LEN = 200
EMBED_DIM = 64
BATCH = 4096

NC, NS, L = 2, 16, 16
NW = NC * NS
BW = BATCH // NW
LC = 2
NCHUNK = MAX_LEN // LC
KB = BW // L
DHI = EMBED_DIM // 8
DV = EMBED_DIM // L


VPAD = 1007616
VHI = VPAD // 128
ABLK = VHI // NW


def _tbody(t4_hbm, trm_hbm, tile, rows_out, isems, osems):
    wid = lax.axis_index("s") * NC + lax.axis_index("c")
    iota = lax.iota(jnp.int32, L)
    n0 = wid * ABLK

    def rd(j, p):
        return pltpu.make_async_copy(
            t4_hbm.at[:, n0 + j], tile.at[p], isems[p])

    def wr(j, p):
        return pltpu.make_async_copy(
            rows_out.at[p, slice(None), pl.ds(0, EMBED_DIM)],
            trm_hbm.at[pl.ds((n0 + j) * 128, 128)], osems[p])

    rd(0, 0).start()

    @pl.loop(0, ABLK, step=2)
    def _blocks(j0):
        for par in range(2):
            j = j0 + par

            @pl.when(j + 1 < ABLK)
            def _pref():
                rd(j + 1, par ^ 1).start()

            @pl.when(j >= 2)
            def _drain():
                wr(j - 2, par).wait()

            rd(j, par).wait()

            @plsc.parallel_loop(0, EMBED_DIM, unroll=4)
            def _cols(d):
                dvec = iota * 0 + d
                dhi = d >> 3
                dlo = d & 7
                for k in range(KB):
                    vals = tile[par, dhi, dlo, pl.ds(k * L, L)]
                    plsc.store_scatter(
                        rows_out.at[par], [iota + k * L, dvec], vals)

            wr(j, par).start()

    wr(ABLK - 2, (ABLK - 2) % 2).wait()
    wr(ABLK - 1, (ABLK - 1) % 2).wait()


def _body(x_hbm, tok_hbm, pos_hbm, out_hbm, pos_v, idx_v, rows_v, obuf,
          isems, gsems, wsems):
    wid = lax.axis_index("s") * NC + lax.axis_index("c")
    b0 = wid * BW
    pltpu.sync_copy(pos_hbm, pos_v)
    iota = lax.iota(jnp.int32, L)

    def idx_fetch(g, p):
        return pltpu.make_async_copy(
            x_hbm.at[pl.ds(g * LC, LC), pl.ds(b0, BW)], idx_v.at[p], isems[p])

    def gathers(g, p):
        del g
        return [pltpu.make_async_copy(
            tok_hbm.at[idx_v.at[p, i]], rows_v.at[p, i], gsems[p])
            for i in range(LC)]

    def write(g, p):
        return pltpu.make_async_copy(
            obuf.at[p, slice(None), slice(None), slice(None), pl.ds(0, BW)],
            out_hbm.at[pl.ds(g * LC, LC), :, wid], wsems[p])

    idx_fetch(0, 0).start()
    idx_fetch(1, 1).start()
    idx_fetch(0, 0).wait()
    for c in gathers(0, 0):
        c.start()

    @pl.loop(0, NCHUNK, step=2)
    def _ring(g0):
        for par in range(2):
            g = g0 + par

            @pl.when(g + 1 < NCHUNK)
            def _launch_next():
                idx_fetch(g + 1, par ^ 1).wait()
                for c in gathers(g + 1, par ^ 1):
                    c.start()

            @pl.when(g >= 2)
            def _drain_write():
                write(g - 2, par).wait()

            for c in gathers(g, par):
                c.wait()

            @pl.when(g + 2 < NCHUNK)
            def _prefetch_idx():
                idx_fetch(g + 2, par).start()

            dhi_vecs = [(iota + q * L) >> 3 for q in range(DV)]
            dlo_vecs = [iota & 7 for _ in range(DV)]
            for i in range(LC):
                lpos = g * LC + i
                pvs = [pos_v[lpos, pl.ds(q * L, L)] for q in range(DV)]

                @plsc.parallel_loop(0, BW, unroll=2)
                def _rows(b):
                    bvec = iota * 0 + b
                    for q in range(DV):
                        vals = rows_v[par, i, b, pl.ds(q * L, L)] + pvs[q]
                        plsc.store_scatter(
                            obuf.at[par, i], [dhi_vecs[q], dlo_vecs[q], bvec],
                            vals)

            write(g, par).start()

    write(NCHUNK - 2, 0).wait()
    write(NCHUNK - 1, 1).wait()


def kernel(x, token_table, pos_table):
    xt = x.T.astype(jnp.int32)
    tp = jnp.pad(token_table, ((0, VPAD - VOCAB), (0, 0)))
    t4 = tp.T.reshape(DHI, 8, VHI, 128).transpose(0, 2, 1, 3)
    trm = pl.kernel(
        _tbody,
        out_type=jax.ShapeDtypeStruct((VPAD, EMBED_DIM), jnp.float32),
        mesh=plsc.VectorSubcoreMesh(core_axis_name="c", subcore_axis_name="s"),
        compiler_params=pltpu.CompilerParams(use_tc_tiling_on_sc=False,
                                             needs_layout_passes=False),
        scratch_types=[
            pltpu.VMEM((2, DHI, 8, 128), jnp.float32),
            pltpu.VMEM((2, 128, EMBED_DIM + 1), jnp.float32),
            [pltpu.SemaphoreType.DMA] * 2,
            [pltpu.SemaphoreType.DMA] * 2,
        ],
    )(t4)
    out5 = pl.kernel(
        _body,
        out_type=jax.ShapeDtypeStruct((MAX_LEN, DHI, NW, 8, BW), jnp.float32),
        mesh=plsc.VectorSubcoreMesh(core_axis_name="c", subcore_axis_name="s"),
        compiler_params=pltpu.CompilerParams(use_tc_tiling_on_sc=False,
                                               needs_layout_passes=False),
        scratch_types=[
            pltpu.VMEM((MAX_LEN, EMBED_DIM), jnp.float32),
            pltpu.VMEM((2, LC, BW), jnp.int32),
            pltpu.VMEM((2, LC, BW, EMBED_DIM), jnp.float32),
            pltpu.VMEM((2, LC, DHI, 8, BW + 1), jnp.float32),
            [pltpu.SemaphoreType.DMA] * 2,
            [pltpu.SemaphoreType.DMA] * 2,
            [pltpu.SemaphoreType.DMA] * 2,
        ],
    )(xt, trm, pos_table)
    return out5.transpose(2, 4, 0, 1, 3).reshape(BATCH, MAX_LEN, EMBED_DIM)


# speedup vs baseline: 3.4229x; 1.0520x over previous
import jax
import jax.numpy as jnp
from jax import lax
from jax.experimental import pallas as pl
from jax.experimental.pallas import tpu as pltpu
from jax.experimental.pallas import tpu_sc as plsc

VOCAB = 1_000_000
MAX_LEN = 200
EMBED_DIM = 64
BATCH = 4096

NC, NS, L = 2, 16, 16
NW = NC * NS
BW = BATCH // NW
LC = 2
NCHUNK = MAX_LEN // LC
KB = BW // L
DHI = EMBED_DIM // 8
DV = EMBED_DIM // L


VPAD = 1015808
VHI = VPAD // 128
ABLK = VHI // NW
NPAIR = ABLK // 2


def _tbody(t4_hbm, trm_hbm, tile, rows_out, isems, osems):
    wid = lax.axis_index("s") * NC + lax.axis_index("c")
    iota = lax.iota(jnp.int32, L)
    p0 = wid * NPAIR

    def rd(j, p):
        return pltpu.make_async_copy(
            t4_hbm.at[:, pl.ds((p0 + j) * 2, 2)], tile.at[p], isems[p])

    def wr(j, p):
        return pltpu.make_async_copy(
            rows_out.at[p, slice(None), pl.ds(0, EMBED_DIM)],
            trm_hbm.at[pl.ds((p0 + j) * 256, 256)], osems[p])

    rd(0, 0).start()

    @pl.loop(0, NPAIR, step=2)
    def _blocks(j0):
        for par in range(2):
            j = j0 + par

            @pl.when(j + 1 < NPAIR)
            def _pref():
                rd(j + 1, par ^ 1).start()

            @pl.when(j >= 2)
            def _drain():
                wr(j - 2, par).wait()

            rd(j, par).wait()

            @plsc.parallel_loop(0, EMBED_DIM, unroll=4)
            def _cols(d):
                dvec = iota * 0 + d
                dhi = d >> 3
                dlo = d & 7
                for v2 in range(2):
                    for k in range(KB):
                        vals = tile[par, dhi, v2, dlo, pl.ds(k * L, L)]
                        plsc.store_scatter(
                            rows_out.at[par],
                            [iota + (v2 * 128 + k * L), dvec], vals)

            wr(j, par).start()

    wr(NPAIR - 2, 0).wait()
    wr(NPAIR - 1, 1).wait()


def _body(x_hbm, tok_hbm, pos_hbm, out_hbm, pos_v, idx_v, rows_v, obuf,
          isems, gsems, wsems):
    wid = lax.axis_index("s") * NC + lax.axis_index("c")
    b0 = wid * BW
    pltpu.sync_copy(pos_hbm, pos_v)
    iota = lax.iota(jnp.int32, L)

    def idx_fetch(g, p):
        return pltpu.make_async_copy(
            x_hbm.at[pl.ds(g * LC, LC), pl.ds(b0, BW)], idx_v.at[p], isems[p])

    def gathers(g, p):
        del g
        return [pltpu.make_async_copy(
            tok_hbm.at[idx_v.at[p, i]], rows_v.at[p, i], gsems[p])
            for i in range(LC)]

    def write(g, p):
        return pltpu.make_async_copy(
            obuf.at[p, slice(None), slice(None), slice(None), pl.ds(0, BW)],
            out_hbm.at[pl.ds(g * LC, LC), :, wid], wsems[p])

    idx_fetch(0, 0).start()
    idx_fetch(1, 1).start()
    idx_fetch(0, 0).wait()
    for c in gathers(0, 0):
        c.start()

    @pl.loop(0, NCHUNK, step=2)
    def _ring(g0):
        for par in range(2):
            g = g0 + par

            @pl.when(g + 1 < NCHUNK)
            def _launch_next():
                idx_fetch(g + 1, par ^ 1).wait()
                for c in gathers(g + 1, par ^ 1):
                    c.start()

            @pl.when(g >= 2)
            def _drain_write():
                write(g - 2, par).wait()

            for c in gathers(g, par):
                c.wait()

            @pl.when(g + 2 < NCHUNK)
            def _prefetch_idx():
                idx_fetch(g + 2, par).start()

            dhi_vecs = [(iota + q * L) >> 3 for q in range(DV)]
            dlo_vecs = [iota & 7 for _ in range(DV)]
            for i in range(LC):
                lpos = g * LC + i
                pvs = [pos_v[lpos, pl.ds(q * L, L)] for q in range(DV)]

                @plsc.parallel_loop(0, BW, unroll=2)
                def _rows(b):
                    bvec = iota * 0 + b
                    for q in range(DV):
                        vals = rows_v[par, i, b, pl.ds(q * L, L)] + pvs[q]
                        plsc.store_scatter(
                            obuf.at[par, i], [dhi_vecs[q], dlo_vecs[q], bvec],
                            vals)

            write(g, par).start()

    write(NCHUNK - 2, 0).wait()
    write(NCHUNK - 1, 1).wait()


def kernel(x, token_table, pos_table):
    xt = x.T.astype(jnp.int32)
    tp = jnp.pad(token_table, ((0, VPAD - VOCAB), (0, 0)))
    t4 = tp.T.reshape(DHI, 8, VHI, 128).transpose(0, 2, 1, 3)
    trm = pl.kernel(
        _tbody,
        out_type=jax.ShapeDtypeStruct((VPAD, EMBED_DIM), jnp.float32),
        mesh=plsc.VectorSubcoreMesh(core_axis_name="c", subcore_axis_name="s"),
        compiler_params=pltpu.CompilerParams(use_tc_tiling_on_sc=False,
                                             needs_layout_passes=False),
        scratch_types=[
            pltpu.VMEM((2, DHI, 2, 8, 128), jnp.float32),
            pltpu.VMEM((2, 256, EMBED_DIM + 1), jnp.float32),
            [pltpu.SemaphoreType.DMA] * 2,
            [pltpu.SemaphoreType.DMA] * 2,
        ],
    )(t4)
    out5 = pl.kernel(
        _body,
        out_type=jax.ShapeDtypeStruct((MAX_LEN, DHI, NW, 8, BW), jnp.float32),
        mesh=plsc.VectorSubcoreMesh(core_axis_name="c", subcore_axis_name="s"),
        compiler_params=pltpu.CompilerParams(use_tc_tiling_on_sc=False,
                                               needs_layout_passes=False),
        scratch_types=[
            pltpu.VMEM((MAX_LEN, EMBED_DIM), jnp.float32),
            pltpu.VMEM((2, LC, BW), jnp.int32),
            pltpu.VMEM((2, LC, BW, EMBED_DIM), jnp.float32),
            pltpu.VMEM((2, LC, DHI, 8, BW + 1), jnp.float32),
            [pltpu.SemaphoreType.DMA] * 2,
            [pltpu.SemaphoreType.DMA] * 2,
            [pltpu.SemaphoreType.DMA] * 2,
        ],
    )(xt, trm, pos_table)
    return out5.transpose(2, 4, 0, 1, 3).reshape(BATCH, MAX_LEN, EMBED_DIM)
